# Initial kernel scaffold; baseline (speedup 1.0000x reference)
import jax
import jax.numpy as jnp
from jax.experimental import pallas as pl


def kernel(positions, hidden_states, ln1_w, qkv_w, o_w, ln_res_w, res_w13, res_w2, ln_post_w, gate_w, ws, w2s):
    raise NotImplementedError("write your pallas kernel here")


# speedup vs baseline: 1.1334x; 1.1334x over previous
import functools

import jax
import jax.numpy as jnp
from jax.experimental import pallas as pl
from jax.experimental.pallas import tpu as pltpu

HIDDEN = 1024
N_HEADS = 16
N_KV = 4
HEAD_DIM = 64
FFN = 1024
E = 8
TOP_K = 2
SEQ = 2048
EPS = 1e-5
THETA = 10000.0

BQ = 512
BS = 512
MOE_T = 256
N_ASSIGN = SEQ * TOP_K
NT = N_ASSIGN // MOE_T + E
PAD_N = NT * MOE_T


def _rms(x):
    return x * jax.lax.rsqrt(jnp.mean(x * x, axis=-1, keepdims=True) + EPS)


def _qkv_kernel(pos_ref, h_ref, ln1_ref, qkvw_ref, q_ref, k_ref, v_ref):
    h = h_ref[...]
    hn = _rms(h) * ln1_ref[0]
    qkv = jnp.dot(hn, qkvw_ref[...].T, preferred_element_type=jnp.float32)
    half = HEAD_DIM // 2
    inv = THETA ** (-(jax.lax.iota(jnp.int32, half).astype(jnp.float32) * 2.0 / HEAD_DIM))
    pos = pos_ref[0].astype(jnp.float32)
    freqs = pos[:, None] * inv[None, :]
    cos = jnp.cos(freqs)
    sin = jnp.sin(freqs)
    scale = HEAD_DIM ** -0.5
    for hh in range(N_HEADS):
        b = hh * HEAD_DIM
        x1 = qkv[:, b:b + half]
        x2 = qkv[:, b + half:b + HEAD_DIM]
        q_ref[hh, :, :half] = (x1 * cos - x2 * sin) * scale
        q_ref[hh, :, half:] = (x2 * cos + x1 * sin) * scale
    for hh in range(N_KV):
        b = N_HEADS * HEAD_DIM + hh * HEAD_DIM
        x1 = qkv[:, b:b + half]
        x2 = qkv[:, b + half:b + HEAD_DIM]
        k_ref[hh, :, :half] = x1 * cos - x2 * sin
        k_ref[hh, :, half:] = x2 * cos + x1 * sin
        vb = (N_HEADS + N_KV) * HEAD_DIM + hh * HEAD_DIM
        v_ref[hh, :, :] = qkv[:, vb:vb + HEAD_DIM]


def _qkv(positions, hidden_states, ln1_w, qkv_w):
    nb = SEQ // BS
    return pl.pallas_call(
        _qkv_kernel,
        grid=(nb,),
        in_specs=[
            pl.BlockSpec((1, BS), lambda i: (0, i)),
            pl.BlockSpec((BS, HIDDEN), lambda i: (i, 0)),
            pl.BlockSpec((1, HIDDEN), lambda i: (0, 0)),
            pl.BlockSpec(((N_HEADS + 2 * N_KV) * HEAD_DIM, HIDDEN), lambda i: (0, 0)),
        ],
        out_specs=(
            pl.BlockSpec((N_HEADS, BS, HEAD_DIM), lambda i: (0, i, 0)),
            pl.BlockSpec((N_KV, BS, HEAD_DIM), lambda i: (0, i, 0)),
            pl.BlockSpec((N_KV, BS, HEAD_DIM), lambda i: (0, i, 0)),
        ),
        out_shape=(
            jax.ShapeDtypeStruct((N_HEADS, SEQ, HEAD_DIM), jnp.float32),
            jax.ShapeDtypeStruct((N_KV, SEQ, HEAD_DIM), jnp.float32),
            jax.ShapeDtypeStruct((N_KV, SEQ, HEAD_DIM), jnp.float32),
        ),
    )(positions.reshape(1, SEQ), hidden_states, ln1_w.reshape(1, HIDDEN), qkv_w)


def _attn_kernel(q_ref, k_ref, v_ref, o_ref):
    i = pl.program_id(1)
    q = q_ref[0]
    k = k_ref[0]
    s = jnp.dot(q, k.T, preferred_element_type=jnp.float32)
    rows = i * BQ + jax.lax.broadcasted_iota(jnp.int32, (BQ, SEQ), 0)
    cols = jax.lax.broadcasted_iota(jnp.int32, (BQ, SEQ), 1)
    s = jnp.where(cols <= rows, s, jnp.float32(-1e9))
    m = jnp.max(s, axis=-1, keepdims=True)
    p = jnp.exp(s - m)
    l = jnp.sum(p, axis=-1, keepdims=True)
    o = jnp.dot(p, v_ref[0], preferred_element_type=jnp.float32)
    o_ref[0] = o / l


def _attention(q, k, v):
    return pl.pallas_call(
        _attn_kernel,
        grid=(N_HEADS, SEQ // BQ),
        in_specs=[
            pl.BlockSpec((1, BQ, HEAD_DIM), lambda h, i: (h, i, 0)),
            pl.BlockSpec((1, SEQ, HEAD_DIM), lambda h, i: (h // (N_HEADS // N_KV), 0, 0)),
            pl.BlockSpec((1, SEQ, HEAD_DIM), lambda h, i: (h // (N_HEADS // N_KV), 0, 0)),
        ],
        out_specs=pl.BlockSpec((1, BQ, HEAD_DIM), lambda h, i: (h, i, 0)),
        out_shape=jax.ShapeDtypeStruct((N_HEADS, SEQ, HEAD_DIM), jnp.float32),
    )(q, k, v)


def _mid_kernel(ao_ref, h0_ref, ow_ref, lnr_ref, w13_ref, w2_ref, lnp_ref,
                gw_ref, rm_ref, hm_ref, probs_ref):
    ao = ao_ref[...]
    ra = h0_ref[...] + jnp.dot(ao, ow_ref[...].T, preferred_element_type=jnp.float32)
    hr = _rms(ra) * lnr_ref[0]
    g13 = jnp.dot(hr, w13_ref[...].T, preferred_element_type=jnp.float32)
    g = g13[:, :HIDDEN]
    act = (g / (1.0 + jnp.exp(-g))) * g13[:, HIDDEN:]
    rm_ref[...] = ra + jnp.dot(act, w2_ref[...].T, preferred_element_type=jnp.float32)
    hm = _rms(ra) * lnp_ref[0]
    hm_ref[...] = hm
    logits = jnp.dot(hm, gw_ref[...].T, preferred_element_type=jnp.float32)
    mx = jnp.max(logits, axis=-1, keepdims=True)
    ex = jnp.exp(logits - mx)
    probs_ref[...] = ex / jnp.sum(ex, axis=-1, keepdims=True)


def _mid(attn_o, hidden_states, o_w, ln_res_w, res_w13, res_w2, ln_post_w, gate_w):
    nb = SEQ // BS
    return pl.pallas_call(
        _mid_kernel,
        grid=(nb,),
        in_specs=[
            pl.BlockSpec((BS, HIDDEN), lambda i: (i, 0)),
            pl.BlockSpec((BS, HIDDEN), lambda i: (i, 0)),
            pl.BlockSpec((HIDDEN, N_HEADS * HEAD_DIM), lambda i: (0, 0)),
            pl.BlockSpec((1, HIDDEN), lambda i: (0, 0)),
            pl.BlockSpec((2 * HIDDEN, HIDDEN), lambda i: (0, 0)),
            pl.BlockSpec((HIDDEN, HIDDEN), lambda i: (0, 0)),
            pl.BlockSpec((1, HIDDEN), lambda i: (0, 0)),
            pl.BlockSpec((E, HIDDEN), lambda i: (0, 0)),
        ],
        out_specs=(
            pl.BlockSpec((BS, HIDDEN), lambda i: (i, 0)),
            pl.BlockSpec((BS, HIDDEN), lambda i: (i, 0)),
            pl.BlockSpec((BS, E), lambda i: (i, 0)),
        ),
        out_shape=(
            jax.ShapeDtypeStruct((SEQ, HIDDEN), jnp.float32),
            jax.ShapeDtypeStruct((SEQ, HIDDEN), jnp.float32),
            jax.ShapeDtypeStruct((SEQ, E), jnp.float32),
        ),
    )(attn_o, hidden_states, o_w, ln_res_w.reshape(1, HIDDEN), res_w13,
      res_w2, ln_post_w.reshape(1, HIDDEN), gate_w)


def _moe_kernel(te_ref, x_ref, ws_ref, w2s_ref, y_ref):
    x = x_ref[...]
    g13 = jnp.dot(x, ws_ref[0].T, preferred_element_type=jnp.float32)
    g = g13[:, :FFN]
    act = (g / (1.0 + jnp.exp(-g))) * g13[:, FFN:]
    y_ref[...] = jnp.dot(act, w2s_ref[0].T, preferred_element_type=jnp.float32)


def _moe_grouped(x_sorted, tile_expert, ws, w2s):
    grid_spec = pltpu.PrefetchScalarGridSpec(
        num_scalar_prefetch=1,
        grid=(NT,),
        in_specs=[
            pl.BlockSpec((MOE_T, HIDDEN), lambda t, te: (t, 0)),
            pl.BlockSpec((1, 2 * FFN, HIDDEN), lambda t, te: (te[t], 0, 0)),
            pl.BlockSpec((1, HIDDEN, FFN), lambda t, te: (te[t], 0, 0)),
        ],
        out_specs=pl.BlockSpec((MOE_T, HIDDEN), lambda t, te: (t, 0)),
    )
    return pl.pallas_call(
        _moe_kernel,
        grid_spec=grid_spec,
        out_shape=jax.ShapeDtypeStruct((PAD_N, HIDDEN), jnp.float32),
    )(tile_expert, x_sorted, ws, w2s)


def _route(probs):
    tw, ti = jax.lax.top_k(probs, TOP_K)
    tw = tw / jnp.sum(tw, axis=-1, keepdims=True)
    flat_e = ti.reshape(-1)
    order = jnp.argsort(flat_e, stable=True)
    counts = jnp.zeros((E,), jnp.int32).at[flat_e].add(1)
    padded = ((counts + MOE_T - 1) // MOE_T) * MOE_T
    start = jnp.concatenate([jnp.zeros((1,), jnp.int32), jnp.cumsum(counts)[:-1]])
    pad_start = jnp.concatenate([jnp.zeros((1,), jnp.int32), jnp.cumsum(padded)[:-1]])
    e_sorted = flat_e[order]
    j = jnp.arange(N_ASSIGN, dtype=jnp.int32)
    p = pad_start[e_sorted] + (j - start[e_sorted])
    row_token = jnp.zeros((PAD_N,), jnp.int32).at[p].set(order // TOP_K)
    pos = jnp.zeros((N_ASSIGN,), jnp.int32).at[order].set(p)
    bounds = jnp.cumsum(padded)
    tile_expert = jnp.minimum(
        jnp.searchsorted(bounds, jnp.arange(NT, dtype=jnp.int32) * MOE_T, side='right'),
        E - 1).astype(jnp.int32)
    return tw, row_token, pos, tile_expert


def kernel(positions, hidden_states, ln1_w, qkv_w, o_w, ln_res_w, res_w13,
           res_w2, ln_post_w, gate_w, ws, w2s):
    q, k, v = _qkv(positions, hidden_states, ln1_w, qkv_w)
    o4 = _attention(q, k, v)
    attn_o = o4.transpose(1, 0, 2).reshape(SEQ, N_HEADS * HEAD_DIM)
    rm, hm, probs = _mid(attn_o, hidden_states, o_w, ln_res_w, res_w13,
                         res_w2, ln_post_w, gate_w)
    tw, row_token, pos, tile_expert = _route(probs)
    x_sorted = jnp.take(hm, row_token, axis=0)
    y = _moe_grouped(x_sorted, tile_expert, ws, w2s)
    pos2 = pos.reshape(SEQ, TOP_K)
    out = rm + tw[:, 0:1] * jnp.take(y, pos2[:, 0], axis=0) \
             + tw[:, 1:2] * jnp.take(y, pos2[:, 1], axis=0)
    return out


# speedup vs baseline: 1.1339x; 1.0004x over previous
import functools

import jax
import jax.numpy as jnp
from jax.experimental import pallas as pl
from jax.experimental.pallas import tpu as pltpu

HIDDEN = 1024
N_HEADS = 16
N_KV = 4
HEAD_DIM = 64
FFN = 1024
E = 8
TOP_K = 2
SEQ = 2048
EPS = 1e-5
THETA = 10000.0

BQ = 512
BS = 512
MOE_T = 256
N_ASSIGN = SEQ * TOP_K
NT = N_ASSIGN // MOE_T + E
PAD_N = NT * MOE_T


def _bdot(a, b):
    return jax.lax.dot_general(
        a.astype(jnp.bfloat16), b.astype(jnp.bfloat16),
        (((1,), (0,)), ((), ())), preferred_element_type=jnp.float32)


def _rms(x):
    return x * jax.lax.rsqrt(jnp.mean(x * x, axis=-1, keepdims=True) + EPS)


def _qkv_kernel(pos_ref, h_ref, ln1_ref, qkvw_ref, q_ref, k_ref, v_ref):
    h = h_ref[...]
    hn = _rms(h) * ln1_ref[0]
    qkv = _bdot(hn, qkvw_ref[...].T)
    half = HEAD_DIM // 2
    inv = THETA ** (-(jax.lax.iota(jnp.int32, half).astype(jnp.float32) * 2.0 / HEAD_DIM))
    pos = pos_ref[0].astype(jnp.float32)
    freqs = pos[:, None] * inv[None, :]
    cos = jnp.cos(freqs)
    sin = jnp.sin(freqs)
    scale = HEAD_DIM ** -0.5
    for hh in range(N_HEADS):
        b = hh * HEAD_DIM
        x1 = qkv[:, b:b + half]
        x2 = qkv[:, b + half:b + HEAD_DIM]
        q_ref[hh, :, :half] = (x1 * cos - x2 * sin) * scale
        q_ref[hh, :, half:] = (x2 * cos + x1 * sin) * scale
    for hh in range(N_KV):
        b = N_HEADS * HEAD_DIM + hh * HEAD_DIM
        x1 = qkv[:, b:b + half]
        x2 = qkv[:, b + half:b + HEAD_DIM]
        k_ref[hh, :, :half] = x1 * cos - x2 * sin
        k_ref[hh, :, half:] = x2 * cos + x1 * sin
        vb = (N_HEADS + N_KV) * HEAD_DIM + hh * HEAD_DIM
        v_ref[hh, :, :] = qkv[:, vb:vb + HEAD_DIM]


def _qkv(positions, hidden_states, ln1_w, qkv_w):
    nb = SEQ // BS
    return pl.pallas_call(
        _qkv_kernel,
        grid=(nb,),
        in_specs=[
            pl.BlockSpec((1, BS), lambda i: (0, i)),
            pl.BlockSpec((BS, HIDDEN), lambda i: (i, 0)),
            pl.BlockSpec((1, HIDDEN), lambda i: (0, 0)),
            pl.BlockSpec(((N_HEADS + 2 * N_KV) * HEAD_DIM, HIDDEN), lambda i: (0, 0)),
        ],
        out_specs=(
            pl.BlockSpec((N_HEADS, BS, HEAD_DIM), lambda i: (0, i, 0)),
            pl.BlockSpec((N_KV, BS, HEAD_DIM), lambda i: (0, i, 0)),
            pl.BlockSpec((N_KV, BS, HEAD_DIM), lambda i: (0, i, 0)),
        ),
        out_shape=(
            jax.ShapeDtypeStruct((N_HEADS, SEQ, HEAD_DIM), jnp.float32),
            jax.ShapeDtypeStruct((N_KV, SEQ, HEAD_DIM), jnp.float32),
            jax.ShapeDtypeStruct((N_KV, SEQ, HEAD_DIM), jnp.float32),
        ),
    )(positions.reshape(1, SEQ), hidden_states, ln1_w.reshape(1, HIDDEN), qkv_w)


def _attn_kernel(q_ref, k_ref, v_ref, o_ref):
    i = pl.program_id(1)
    q = q_ref[0]
    k = k_ref[0]
    s = _bdot(q, k.T)
    rows = i * BQ + jax.lax.broadcasted_iota(jnp.int32, (BQ, SEQ), 0)
    cols = jax.lax.broadcasted_iota(jnp.int32, (BQ, SEQ), 1)
    s = jnp.where(cols <= rows, s, jnp.float32(-1e9))
    m = jnp.max(s, axis=-1, keepdims=True)
    p = jnp.exp(s - m)
    l = jnp.sum(p, axis=-1, keepdims=True)
    o = _bdot(p, v_ref[0])
    o_ref[0] = o / l


def _attention(q, k, v):
    return pl.pallas_call(
        _attn_kernel,
        grid=(N_HEADS, SEQ // BQ),
        in_specs=[
            pl.BlockSpec((1, BQ, HEAD_DIM), lambda h, i: (h, i, 0)),
            pl.BlockSpec((1, SEQ, HEAD_DIM), lambda h, i: (h // (N_HEADS // N_KV), 0, 0)),
            pl.BlockSpec((1, SEQ, HEAD_DIM), lambda h, i: (h // (N_HEADS // N_KV), 0, 0)),
        ],
        out_specs=pl.BlockSpec((1, BQ, HEAD_DIM), lambda h, i: (h, i, 0)),
        out_shape=jax.ShapeDtypeStruct((N_HEADS, SEQ, HEAD_DIM), jnp.float32),
    )(q, k, v)


def _mid_kernel(ao_ref, h0_ref, ow_ref, lnr_ref, w13_ref, w2_ref, lnp_ref,
                gw_ref, rm_ref, hm_ref, probs_ref):
    ao = ao_ref[...]
    ra = h0_ref[...] + _bdot(ao, ow_ref[...].T)
    hr = _rms(ra) * lnr_ref[0]
    g13 = _bdot(hr, w13_ref[...].T)
    g = g13[:, :HIDDEN]
    act = (g / (1.0 + jnp.exp(-g))) * g13[:, HIDDEN:]
    rm_ref[...] = ra + _bdot(act, w2_ref[...].T)
    hm = _rms(ra) * lnp_ref[0]
    hm_ref[...] = hm
    logits = jnp.dot(hm, gw_ref[...].T, preferred_element_type=jnp.float32)
    mx = jnp.max(logits, axis=-1, keepdims=True)
    ex = jnp.exp(logits - mx)
    probs_ref[...] = ex / jnp.sum(ex, axis=-1, keepdims=True)


def _mid(attn_o, hidden_states, o_w, ln_res_w, res_w13, res_w2, ln_post_w, gate_w):
    nb = SEQ // BS
    return pl.pallas_call(
        _mid_kernel,
        grid=(nb,),
        in_specs=[
            pl.BlockSpec((BS, HIDDEN), lambda i: (i, 0)),
            pl.BlockSpec((BS, HIDDEN), lambda i: (i, 0)),
            pl.BlockSpec((HIDDEN, N_HEADS * HEAD_DIM), lambda i: (0, 0)),
            pl.BlockSpec((1, HIDDEN), lambda i: (0, 0)),
            pl.BlockSpec((2 * HIDDEN, HIDDEN), lambda i: (0, 0)),
            pl.BlockSpec((HIDDEN, HIDDEN), lambda i: (0, 0)),
            pl.BlockSpec((1, HIDDEN), lambda i: (0, 0)),
            pl.BlockSpec((E, HIDDEN), lambda i: (0, 0)),
        ],
        out_specs=(
            pl.BlockSpec((BS, HIDDEN), lambda i: (i, 0)),
            pl.BlockSpec((BS, HIDDEN), lambda i: (i, 0)),
            pl.BlockSpec((BS, E), lambda i: (i, 0)),
        ),
        out_shape=(
            jax.ShapeDtypeStruct((SEQ, HIDDEN), jnp.float32),
            jax.ShapeDtypeStruct((SEQ, HIDDEN), jnp.float32),
            jax.ShapeDtypeStruct((SEQ, E), jnp.float32),
        ),
    )(attn_o, hidden_states, o_w, ln_res_w.reshape(1, HIDDEN), res_w13,
      res_w2, ln_post_w.reshape(1, HIDDEN), gate_w)


def _moe_kernel(te_ref, x_ref, ws_ref, w2s_ref, y_ref):
    x = x_ref[...]
    g13 = _bdot(x, ws_ref[0].T)
    g = g13[:, :FFN]
    act = (g / (1.0 + jnp.exp(-g))) * g13[:, FFN:]
    y_ref[...] = _bdot(act, w2s_ref[0].T)


def _moe_grouped(x_sorted, tile_expert, ws, w2s):
    grid_spec = pltpu.PrefetchScalarGridSpec(
        num_scalar_prefetch=1,
        grid=(NT,),
        in_specs=[
            pl.BlockSpec((MOE_T, HIDDEN), lambda t, te: (t, 0)),
            pl.BlockSpec((1, 2 * FFN, HIDDEN), lambda t, te: (te[t], 0, 0)),
            pl.BlockSpec((1, HIDDEN, FFN), lambda t, te: (te[t], 0, 0)),
        ],
        out_specs=pl.BlockSpec((MOE_T, HIDDEN), lambda t, te: (t, 0)),
    )
    return pl.pallas_call(
        _moe_kernel,
        grid_spec=grid_spec,
        out_shape=jax.ShapeDtypeStruct((PAD_N, HIDDEN), jnp.float32),
    )(tile_expert, x_sorted, ws, w2s)


def _route(probs):
    tw, ti = jax.lax.top_k(probs, TOP_K)
    tw = tw / jnp.sum(tw, axis=-1, keepdims=True)
    flat_e = ti.reshape(-1)
    order = jnp.argsort(flat_e, stable=True)
    counts = jnp.zeros((E,), jnp.int32).at[flat_e].add(1)
    padded = ((counts + MOE_T - 1) // MOE_T) * MOE_T
    start = jnp.concatenate([jnp.zeros((1,), jnp.int32), jnp.cumsum(counts)[:-1]])
    pad_start = jnp.concatenate([jnp.zeros((1,), jnp.int32), jnp.cumsum(padded)[:-1]])
    e_sorted = flat_e[order]
    j = jnp.arange(N_ASSIGN, dtype=jnp.int32)
    p = pad_start[e_sorted] + (j - start[e_sorted])
    row_token = jnp.zeros((PAD_N,), jnp.int32).at[p].set(order // TOP_K)
    pos = jnp.zeros((N_ASSIGN,), jnp.int32).at[order].set(p)
    bounds = jnp.cumsum(padded)
    tile_expert = jnp.minimum(
        jnp.searchsorted(bounds, jnp.arange(NT, dtype=jnp.int32) * MOE_T, side='right'),
        E - 1).astype(jnp.int32)
    return tw, row_token, pos, tile_expert


def kernel(positions, hidden_states, ln1_w, qkv_w, o_w, ln_res_w, res_w13,
           res_w2, ln_post_w, gate_w, ws, w2s):
    q, k, v = _qkv(positions, hidden_states, ln1_w, qkv_w)
    o4 = _attention(q, k, v)
    attn_o = o4.transpose(1, 0, 2).reshape(SEQ, N_HEADS * HEAD_DIM)
    rm, hm, probs = _mid(attn_o, hidden_states, o_w, ln_res_w, res_w13,
                         res_w2, ln_post_w, gate_w)
    tw, row_token, pos, tile_expert = _route(probs)
    x_sorted = jnp.take(hm, row_token, axis=0)
    y = _moe_grouped(x_sorted, tile_expert, ws, w2s)
    pos2 = pos.reshape(SEQ, TOP_K)
    out = rm + tw[:, 0:1] * jnp.take(y, pos2[:, 0], axis=0) \
             + tw[:, 1:2] * jnp.take(y, pos2[:, 1], axis=0)
    return out
